# Initial kernel scaffold; baseline (speedup 1.0000x reference)
import jax
import jax.numpy as jnp
from jax.experimental import pallas as pl


def kernel(z, edge_index):
    raise NotImplementedError("write your pallas kernel here")


# speedup vs baseline: 3.1693x; 3.1693x over previous
import functools

import jax
import jax.numpy as jnp
from jax import lax
from jax.experimental import pallas as pl
from jax.experimental.pallas import tpu as pltpu
from jax.experimental.pallas import tpu_sc as plsc

D = 128
E = 320000
NC = 2
NS = 16
NW = NC * NS
E_W = E // NW
E_BLK = 400
N_BLK = E_W // E_BLK


def _dot_body(z_hbm, u_hbm, v_hbm, out_hbm,
              uidx_v, vidx_v, zu_v, zv_v, out_v, sem_u, sem_v):
    wid = lax.axis_index("s") * NC + lax.axis_index("c")
    base = wid * E_W

    def block(b, carry):
        off = base + b * E_BLK
        pltpu.sync_copy(u_hbm.at[pl.ds(off, E_BLK)], uidx_v)
        pltpu.sync_copy(v_hbm.at[pl.ds(off, E_BLK)], vidx_v)
        cu = pltpu.async_copy(z_hbm.at[uidx_v], zu_v, sem_u)
        cv = pltpu.async_copy(z_hbm.at[vidx_v], zv_v, sem_v)
        cu.wait()
        cv.wait()

        lane = lax.iota(jnp.int32, 16)

        def group(g, c):
            res = jnp.zeros((16,), jnp.float32)
            for j in range(16):
                e = g * 16 + j
                acc = zu_v[e, pl.ds(0, 16)] * zv_v[e, pl.ds(0, 16)]
                for ch in range(1, D // 16):
                    acc = acc + (zu_v[e, pl.ds(ch * 16, 16)]
                                 * zv_v[e, pl.ds(ch * 16, 16)])
                res = jnp.where(lane == j, jnp.sum(acc), res)
            out_v[pl.ds(g * 16, 16)] = res
            return c

        lax.fori_loop(0, E_BLK // 16, group, 0, unroll=False)
        pltpu.sync_copy(out_v, out_hbm.at[pl.ds(off, E_BLK)])
        return carry

    lax.fori_loop(0, N_BLK, block, 0, unroll=False)


@functools.partial(jax.jit, donate_argnums=())
def _dot_sc(z, u, v):
    mesh = plsc.VectorSubcoreMesh(core_axis_name="c", subcore_axis_name="s")
    return pl.kernel(
        _dot_body,
        mesh=mesh,
        compiler_params=pltpu.CompilerParams(needs_layout_passes=False),
        out_type=jax.ShapeDtypeStruct((E,), jnp.float32),
        scratch_types=[
            pltpu.VMEM((E_BLK,), jnp.int32),
            pltpu.VMEM((E_BLK,), jnp.int32),
            pltpu.VMEM((E_BLK, D), jnp.float32),
            pltpu.VMEM((E_BLK, D), jnp.float32),
            pltpu.VMEM((E_BLK,), jnp.float32),
            pltpu.SemaphoreType.DMA,
            pltpu.SemaphoreType.DMA,
        ],
    )(z, u, v)


def kernel(z, edge_index):
    u = edge_index[0].astype(jnp.int32)
    v = edge_index[1].astype(jnp.int32)
    return _dot_sc(z, u, v)
